# Initial kernel scaffold; baseline (speedup 1.0000x reference)
import jax
import jax.numpy as jnp
from jax.experimental import pallas as pl


def kernel(x, router_w, w1, w2, w3):
    raise NotImplementedError("write your pallas kernel here")


# speedup vs baseline: 1.0500x; 1.0500x over previous
import functools

import jax
import jax.numpy as jnp
from jax import lax
from jax.experimental import pallas as pl
from jax.experimental.pallas import tpu as pltpu
from jax.experimental.pallas import tpu_sc as plsc

N = 2048
D = 1024
H = 2816
E = 8
TOPK = 2

T = 256
NB = 24
A_PAD = NB * T
HB = 256
NH = H // HB

NC, NS = 2, 16
NW = NC * NS


def _router_body(x_ref, rw_ref, w_out, se_out, p0_out, p1_out,
                 g0_out, g1_out, ts_out, be_out):
    x = x_ref[...]
    rw = rw_ref[...]
    logits = lax.dot_general(x, rw, (((1,), (1,)), ((), ())),
                             preferred_element_type=jnp.float32)

    e_iota = lax.broadcasted_iota(jnp.int32, (N, E), 1)
    m0 = jnp.max(logits, axis=1, keepdims=True)
    i0 = jnp.min(jnp.where(logits == m0, e_iota, E), axis=1, keepdims=True)
    l2 = jnp.where(e_iota == i0, jnp.float32(-1e30), logits)
    m1 = jnp.max(l2, axis=1, keepdims=True)
    i1 = jnp.min(jnp.where(l2 == m1, e_iota, E), axis=1, keepdims=True)

    t = jnp.exp(m1 - m0)
    w0 = 1.0 / (1.0 + t)
    w1g = t / (1.0 + t)
    w_out[...] = jnp.concatenate([w0, w1g], axis=1)
    se_out[...] = jnp.concatenate([i0, i1], axis=1)
    g0_out[...] = w0 + jnp.zeros((N, 16), jnp.float32)
    g1_out[...] = w1g + jnp.zeros((N, 16), jnp.float32)

    oh0 = (e_iota == i0).astype(jnp.float32)
    oh1 = (e_iota == i1).astype(jnp.float32)
    C = 512
    r_i = lax.broadcasted_iota(jnp.int32, (C, C), 0)
    c_i = lax.broadcasted_iota(jnp.int32, (C, C), 1)
    tri = (c_i < r_i).astype(jnp.float32)

    def cumsum_excl(oh):
        run = jnp.zeros((1, E), jnp.float32)
        outs = []
        for c in range(N // C):
            blk = oh[c * C:(c + 1) * C, :]
            outs.append(jnp.dot(tri, blk, preferred_element_type=jnp.float32)
                        + run)
            run = run + jnp.sum(blk, axis=0, keepdims=True)
        return jnp.concatenate(outs, axis=0), run

    r0, tot0 = cumsum_excl(oh0)
    r1, tot1 = cumsum_excl(oh1)
    rank0 = jnp.sum(r0 * oh0, axis=1, keepdims=True)
    rank1 = jnp.sum((r1 + tot0) * oh1, axis=1, keepdims=True)

    counts = tot0 + tot1
    padded = jnp.floor((counts + (T - 1)) * (1.0 / T)) * T
    er = lax.broadcasted_iota(jnp.int32, (E, E), 0)
    ec = lax.broadcasted_iota(jnp.int32, (E, E), 1)
    tri8 = (er < ec).astype(jnp.float32)
    offs = jnp.dot(padded, tri8, preferred_element_type=jnp.float32)

    pos0 = jnp.sum(oh0 * offs, axis=1, keepdims=True) + rank0
    pos1 = jnp.sum(oh1 * offs, axis=1, keepdims=True) + rank1
    p0_out[...] = pos0.astype(jnp.int32)
    p1_out[...] = pos1.astype(jnp.int32)

    n_col = lax.broadcasted_iota(jnp.int32, (N, 1), 0).astype(jnp.float32)
    SC = 512
    for c in range(A_PAD // SC):
        slots = (lax.broadcasted_iota(jnp.int32, (1, SC), 1) + (c * SC)).astype(jnp.float32)
        hit0 = pos0 == slots
        hit1 = pos1 == slots
        tsc = jnp.sum(jnp.where(hit0, n_col, 0.0)
                      + jnp.where(hit1, n_col, 0.0), axis=0, keepdims=True)
        ts_out[0:1, c * SC:(c + 1) * SC] = tsc.astype(jnp.int32)

    b_start = (lax.broadcasted_iota(jnp.int32, (NB, 1), 0) * T).astype(jnp.float32)
    ind = (b_start >= offs) & (b_start < offs + padded)
    e_row = lax.broadcasted_iota(jnp.int32, (NB, E), 1).astype(jnp.float32)
    be = jnp.sum(jnp.where(ind, e_row, 0.0), axis=1, keepdims=True)
    be_out[...] = be.astype(jnp.int32)


def _router_call(x2d, router_w):
    return pl.pallas_call(
        _router_body,
        out_shape=[
            jax.ShapeDtypeStruct((N, TOPK), jnp.float32),
            jax.ShapeDtypeStruct((N, TOPK), jnp.int32),
            jax.ShapeDtypeStruct((N, 1), jnp.int32),
            jax.ShapeDtypeStruct((N, 1), jnp.int32),
            jax.ShapeDtypeStruct((N, 16), jnp.float32),
            jax.ShapeDtypeStruct((N, 16), jnp.float32),
            jax.ShapeDtypeStruct((1, A_PAD), jnp.int32),
            jax.ShapeDtypeStruct((NB, 1), jnp.int32),
        ],
    )(x2d, router_w)


def _gather_body(x_hbm, ts_hbm, xs_hbm, idx_v, rows_v, sem):
    wid = lax.axis_index("s") * NC + lax.axis_index("c")
    per = A_PAD // NW
    base = wid * per
    pltpu.sync_copy(ts_hbm.at[pl.ds(base, per)], idx_v)
    for c in range(per // 64):
        pltpu.async_copy(
            x_hbm.at[idx_v.at[pl.ds(c * 64, 64)]], rows_v, sem).wait()
        pltpu.sync_copy(rows_v, xs_hbm.at[pl.ds(base + c * 64, 64)])


def _gather_call(x2d, token_sorted):
    f = pl.kernel(
        _gather_body,
        out_type=jax.ShapeDtypeStruct((A_PAD, D), jnp.float32),
        mesh=plsc.VectorSubcoreMesh(core_axis_name="c", subcore_axis_name="s"),
        scratch_types=[
            pltpu.VMEM((A_PAD // NW,), jnp.int32),
            pltpu.VMEM((64, D), jnp.float32),
            pltpu.SemaphoreType.DMA,
        ],
    )
    return f(x2d, token_sorted)


def _expert_body(be_ref, xs_ref, w1_ref, w3_ref, w2_ref, ys_ref, acc_ref):
    j = pl.program_id(1)
    x = xs_ref[...]
    w1e = jnp.squeeze(w1_ref[...], 0)
    w3e = jnp.squeeze(w3_ref[...], 0)
    w2e = jnp.squeeze(w2_ref[...], 0)
    a = lax.dot_general(x, w1e, (((1,), (1,)), ((), ())),
                        preferred_element_type=jnp.float32)
    b = lax.dot_general(x, w3e, (((1,), (1,)), ((), ())),
                        preferred_element_type=jnp.float32)
    h = a * jax.nn.sigmoid(a) * b
    contrib = lax.dot_general(h, w2e, (((1,), (1,)), ((), ())),
                              preferred_element_type=jnp.float32)

    @pl.when(j == 0)
    def _():
        acc_ref[...] = contrib

    @pl.when(j > 0)
    def _():
        acc_ref[...] += contrib

    @pl.when(j == NH - 1)
    def _():
        ys_ref[...] = acc_ref[...]


def _expert_call(xs, w1, w3, w2, block_expert):
    grid_spec = pltpu.PrefetchScalarGridSpec(
        num_scalar_prefetch=1,
        grid=(NB, NH),
        in_specs=[
            pl.BlockSpec((T, D), lambda g, j, be: (g, 0)),
            pl.BlockSpec((1, HB, D), lambda g, j, be: (be[g], j, 0)),
            pl.BlockSpec((1, HB, D), lambda g, j, be: (be[g], j, 0)),
            pl.BlockSpec((1, D, HB), lambda g, j, be: (be[g], 0, j)),
        ],
        out_specs=pl.BlockSpec((T, D), lambda g, j, be: (g, 0)),
        scratch_shapes=[pltpu.VMEM((T, D), jnp.float32)],
    )
    return pl.pallas_call(
        _expert_body,
        grid_spec=grid_spec,
        out_shape=jax.ShapeDtypeStruct((A_PAD, D), jnp.float32),
        compiler_params=pltpu.CompilerParams(
            dimension_semantics=("arbitrary", "arbitrary")),
    )(block_expert, xs, w1, w3, w2)


def _combine_body(ys_hbm, p0_hbm, p1_hbm, g0_hbm, g1_hbm, out_hbm,
                  i0_v, i1_v, g0_v, g1_v, a_v, b_v, o_v, sem):
    wid = lax.axis_index("s") * NC + lax.axis_index("c")
    per = N // NW
    base = wid * per
    pltpu.sync_copy(p0_hbm.at[pl.ds(base, per)], i0_v)
    pltpu.sync_copy(p1_hbm.at[pl.ds(base, per)], i1_v)
    pltpu.sync_copy(g0_hbm.at[pl.ds(base, per)], g0_v)
    pltpu.sync_copy(g1_hbm.at[pl.ds(base, per)], g1_v)
    for c in range(per // 16):
        ca = pltpu.async_copy(ys_hbm.at[i0_v.at[pl.ds(c * 16, 16)]], a_v, sem)
        cb = pltpu.async_copy(ys_hbm.at[i1_v.at[pl.ds(c * 16, 16)]], b_v, sem)
        ca.wait()
        cb.wait()

        def row(t, _):
            tt = c * 16 + t
            g0b = g0_v[tt, :]
            g1b = g1_v[tt, :]
            for v in range(D // 16):
                s = pl.ds(v * 16, 16)
                o_v[t, s] = g0b * a_v[t, s] + g1b * b_v[t, s]
            return 0

        lax.fori_loop(0, 16, row, 0)
        pltpu.sync_copy(o_v, out_hbm.at[pl.ds(base + c * 16, 16)])


def _combine_call(ys, pos0, pos1, gate0, gate1):
    f = pl.kernel(
        _combine_body,
        out_type=jax.ShapeDtypeStruct((N, D), jnp.float32),
        mesh=plsc.VectorSubcoreMesh(core_axis_name="c", subcore_axis_name="s"),
        scratch_types=[
            pltpu.VMEM((N // NW,), jnp.int32),
            pltpu.VMEM((N // NW,), jnp.int32),
            pltpu.VMEM((N // NW, 16), jnp.float32),
            pltpu.VMEM((N // NW, 16), jnp.float32),
            pltpu.VMEM((16, D), jnp.float32),
            pltpu.VMEM((16, D), jnp.float32),
            pltpu.VMEM((16, D), jnp.float32),
            pltpu.SemaphoreType.DMA,
        ],
    )
    return f(ys, pos0, pos1, gate0, gate1)


def kernel(x, router_w, w1, w2, w3):
    x2d = x.reshape(N, D)
    (w_top, se, p0, p1, g0, g1, ts, be) = _router_call(x2d, router_w)
    xs = _gather_call(x2d, ts.reshape(A_PAD))
    ys = _expert_call(xs, w1, w3, w2, be.reshape(NB))
    out = _combine_call(ys, p0.reshape(N), p1.reshape(N), g0, g1)
    return (out.reshape(1, N, D),
            w_top.reshape(1, N, TOPK),
            se.reshape(1, N, TOPK))


# speedup vs baseline: 1.2941x; 1.2325x over previous
import functools

import jax
import jax.numpy as jnp
from jax import lax
from jax.experimental import pallas as pl
from jax.experimental.pallas import tpu as pltpu
from jax.experimental.pallas import tpu_sc as plsc

N = 2048
D = 1024
H = 2816
E = 8
TOPK = 2

T = 256
NB = 24
A_PAD = NB * T
HB = 256
NH = H // HB

NC, NS = 2, 16
NW = NC * NS


def _router_body(x_ref, rw_ref, w_out, se_out, p0_out, p1_out,
                 g0_out, g1_out, be_out):
    x = x_ref[...]
    rw = rw_ref[...]
    logits = lax.dot_general(x, rw, (((1,), (1,)), ((), ())),
                             preferred_element_type=jnp.float32)

    e_iota = lax.broadcasted_iota(jnp.int32, (N, E), 1)
    m0 = jnp.max(logits, axis=1, keepdims=True)
    i0 = jnp.min(jnp.where(logits == m0, e_iota, E), axis=1, keepdims=True)
    l2 = jnp.where(e_iota == i0, jnp.float32(-1e30), logits)
    m1 = jnp.max(l2, axis=1, keepdims=True)
    i1 = jnp.min(jnp.where(l2 == m1, e_iota, E), axis=1, keepdims=True)

    t = jnp.exp(m1 - m0)
    w0 = 1.0 / (1.0 + t)
    w1g = t / (1.0 + t)
    w_out[...] = jnp.concatenate([w0, w1g], axis=1)
    se_out[...] = jnp.concatenate([i0, i1], axis=1)
    g0_out[...] = w0 + jnp.zeros((N, 16), jnp.float32)
    g1_out[...] = w1g + jnp.zeros((N, 16), jnp.float32)

    oh0 = (e_iota == i0).astype(jnp.float32)
    oh1 = (e_iota == i1).astype(jnp.float32)
    C = 512
    r_i = lax.broadcasted_iota(jnp.int32, (C, C), 0)
    c_i = lax.broadcasted_iota(jnp.int32, (C, C), 1)
    tri = (c_i < r_i).astype(jnp.float32)

    def cumsum_excl(oh):
        run = jnp.zeros((1, E), jnp.float32)
        outs = []
        for c in range(N // C):
            blk = oh[c * C:(c + 1) * C, :]
            outs.append(jnp.dot(tri, blk, preferred_element_type=jnp.float32)
                        + run)
            run = run + jnp.sum(blk, axis=0, keepdims=True)
        return jnp.concatenate(outs, axis=0), run

    r0, tot0 = cumsum_excl(oh0)
    r1, tot1 = cumsum_excl(oh1)
    rank0 = jnp.sum(r0 * oh0, axis=1, keepdims=True)
    rank1 = jnp.sum((r1 + tot0) * oh1, axis=1, keepdims=True)

    counts = tot0 + tot1
    padded = jnp.floor((counts + (T - 1)) * (1.0 / T)) * T
    er = lax.broadcasted_iota(jnp.int32, (E, E), 0)
    ec = lax.broadcasted_iota(jnp.int32, (E, E), 1)
    tri8 = (er < ec).astype(jnp.float32)
    offs = jnp.dot(padded, tri8, preferred_element_type=jnp.float32)

    pos0 = jnp.sum(oh0 * offs, axis=1, keepdims=True) + rank0
    pos1 = jnp.sum(oh1 * offs, axis=1, keepdims=True) + rank1
    p0_out[...] = pos0.astype(jnp.int32)
    p1_out[...] = pos1.astype(jnp.int32)

    b_start = (lax.broadcasted_iota(jnp.int32, (NB, 1), 0) * T).astype(jnp.float32)
    ind = (b_start >= offs) & (b_start < offs + padded)
    e_row = lax.broadcasted_iota(jnp.int32, (NB, E), 1).astype(jnp.float32)
    be = jnp.sum(jnp.where(ind, e_row, 0.0), axis=1, keepdims=True)
    be_out[...] = be.astype(jnp.int32)


def _router_call(x2d, router_w):
    return pl.pallas_call(
        _router_body,
        out_shape=[
            jax.ShapeDtypeStruct((N, TOPK), jnp.float32),
            jax.ShapeDtypeStruct((N, TOPK), jnp.int32),
            jax.ShapeDtypeStruct((N, 1), jnp.int32),
            jax.ShapeDtypeStruct((N, 1), jnp.int32),
            jax.ShapeDtypeStruct((N, 16), jnp.float32),
            jax.ShapeDtypeStruct((N, 16), jnp.float32),
            jax.ShapeDtypeStruct((NB, 1), jnp.int32),
        ],
    )(x2d, router_w)


_GCH = 32
_APT = 2 * N // NW


def _gather_body(x_hbm, pos_hbm, xs_hbm,
                 idx0_v, idx1_v, rows0_v, rows1_v, sem):
    wid = lax.axis_index("s") * NC + lax.axis_index("c")
    abase = wid * _APT
    rbase = (wid % (NW // 2)) * _APT
    idxs = (idx0_v, idx1_v)
    rows = (rows0_v, rows1_v)
    prev = None
    for c in range(_APT // _GCH):
        b = c % 2
        pltpu.sync_copy(pos_hbm.at[pl.ds(abase + c * _GCH, _GCH)], idxs[b])
        pltpu.sync_copy(x_hbm.at[pl.ds(rbase + c * _GCH, _GCH)], rows[b])
        if prev is not None:
            prev.wait()
        prev = pltpu.async_copy(rows[b], xs_hbm.at[idxs[b]], sem)
    prev.wait()


def _gather_call(x2d, pos_all):
    f = pl.kernel(
        _gather_body,
        out_type=jax.ShapeDtypeStruct((A_PAD, D), jnp.float32),
        mesh=plsc.VectorSubcoreMesh(core_axis_name="c", subcore_axis_name="s"),
        scratch_types=[
            pltpu.VMEM((_GCH,), jnp.int32),
            pltpu.VMEM((_GCH,), jnp.int32),
            pltpu.VMEM((_GCH, D), jnp.float32),
            pltpu.VMEM((_GCH, D), jnp.float32),
            pltpu.SemaphoreType.DMA,
        ],
    )
    return f(x2d, pos_all)


def _expert_body(be_ref, xs_ref, w1_ref, w3_ref, w2_ref, ys_ref):
    j = pl.program_id(0)
    g = pl.program_id(1)
    x = xs_ref[...]
    w1e = jnp.squeeze(w1_ref[...], 0)
    w3e = jnp.squeeze(w3_ref[...], 0)
    w2e = jnp.squeeze(w2_ref[...], 0)
    a = lax.dot_general(x, w1e, (((1,), (1,)), ((), ())),
                        preferred_element_type=jnp.float32)
    b = lax.dot_general(x, w3e, (((1,), (1,)), ((), ())),
                        preferred_element_type=jnp.float32)
    h = a * jax.nn.sigmoid(a) * b
    contrib = lax.dot_general(h, w2e, (((1,), (1,)), ((), ())),
                              preferred_element_type=jnp.float32)
    rows = pl.ds(g * T, T)

    @pl.when(j == 0)
    def _():
        ys_ref[rows, :] = contrib

    @pl.when(j > 0)
    def _():
        ys_ref[rows, :] += contrib


def _expert_call(xs, w1, w3, w2, block_expert):
    grid_spec = pltpu.PrefetchScalarGridSpec(
        num_scalar_prefetch=1,
        grid=(NH, NB),
        in_specs=[
            pl.BlockSpec((T, D), lambda j, g, be: (g, 0)),
            pl.BlockSpec((1, HB, D), lambda j, g, be: (be[g], j, 0)),
            pl.BlockSpec((1, HB, D), lambda j, g, be: (be[g], j, 0)),
            pl.BlockSpec((1, D, HB), lambda j, g, be: (be[g], 0, j)),
        ],
        out_specs=pl.BlockSpec((A_PAD, D), lambda j, g, be: (0, 0)),
    )
    return pl.pallas_call(
        _expert_body,
        grid_spec=grid_spec,
        out_shape=jax.ShapeDtypeStruct((A_PAD, D), jnp.float32),
        compiler_params=pltpu.CompilerParams(
            dimension_semantics=("arbitrary", "arbitrary")),
    )(block_expert, xs, w1, w3, w2)


def _combine_body(ys_hbm, p0_hbm, p1_hbm, g0_hbm, g1_hbm, out_hbm,
                  i0_v, i1_v, g0_v, g1_v, a_v, b_v, o_v, sem):
    wid = lax.axis_index("s") * NC + lax.axis_index("c")
    per = N // NW
    base = wid * per
    pltpu.sync_copy(p0_hbm.at[pl.ds(base, per)], i0_v)
    pltpu.sync_copy(p1_hbm.at[pl.ds(base, per)], i1_v)
    pltpu.sync_copy(g0_hbm.at[pl.ds(base, per)], g0_v)
    pltpu.sync_copy(g1_hbm.at[pl.ds(base, per)], g1_v)
    for c in range(per // 16):
        ca = pltpu.async_copy(ys_hbm.at[i0_v.at[pl.ds(c * 16, 16)]], a_v, sem)
        cb = pltpu.async_copy(ys_hbm.at[i1_v.at[pl.ds(c * 16, 16)]], b_v, sem)
        ca.wait()
        cb.wait()

        def row(t, _):
            tt = c * 16 + t
            g0b = g0_v[tt, :]
            g1b = g1_v[tt, :]
            for v in range(D // 16):
                s = pl.ds(v * 16, 16)
                o_v[t, s] = g0b * a_v[t, s] + g1b * b_v[t, s]
            return 0

        lax.fori_loop(0, 16, row, 0)
        pltpu.sync_copy(o_v, out_hbm.at[pl.ds(base + c * 16, 16)])


def _combine_call(ys, pos0, pos1, gate0, gate1):
    f = pl.kernel(
        _combine_body,
        out_type=jax.ShapeDtypeStruct((N, D), jnp.float32),
        mesh=plsc.VectorSubcoreMesh(core_axis_name="c", subcore_axis_name="s"),
        scratch_types=[
            pltpu.VMEM((N // NW,), jnp.int32),
            pltpu.VMEM((N // NW,), jnp.int32),
            pltpu.VMEM((N // NW, 16), jnp.float32),
            pltpu.VMEM((N // NW, 16), jnp.float32),
            pltpu.VMEM((16, D), jnp.float32),
            pltpu.VMEM((16, D), jnp.float32),
            pltpu.VMEM((16, D), jnp.float32),
            pltpu.SemaphoreType.DMA,
        ],
    )
    return f(ys, pos0, pos1, gate0, gate1)


def kernel(x, router_w, w1, w2, w3):
    x2d = x.reshape(N, D)
    (w_top, se, p0, p1, g0, g1, be) = _router_call(x2d, router_w)
    pos_all = jnp.concatenate([p0.reshape(N), p1.reshape(N)])
    xs = _gather_call(x2d, pos_all)
    ys = _expert_call(xs, w1, w3, w2, be.reshape(NB))
    out = _combine_call(ys, p0.reshape(N), p1.reshape(N), g0, g1)
    return (out.reshape(1, N, D),
            w_top.reshape(1, N, TOPK),
            se.reshape(1, N, TOPK))


# speedup vs baseline: 1.5396x; 1.1897x over previous
import functools

import jax
import jax.numpy as jnp
from jax import lax
from jax.experimental import pallas as pl
from jax.experimental.pallas import tpu as pltpu
from jax.experimental.pallas import tpu_sc as plsc

N = 2048
D = 1024
H = 2816
E = 8
TOPK = 2

T = 128
NB = 40
A_PAD = NB * T
HB = 1408
NH = H // HB

NC, NS = 2, 16
NW = NC * NS


def _router_body(x_ref, rw_ref, w_out, se_out, p0_out, p1_out,
                 g0_out, g1_out, be_out):
    x = x_ref[...]
    rw = rw_ref[...]
    logits = lax.dot_general(x, rw, (((1,), (1,)), ((), ())),
                             preferred_element_type=jnp.float32)

    e_iota = lax.broadcasted_iota(jnp.int32, (N, E), 1)
    m0 = jnp.max(logits, axis=1, keepdims=True)
    i0 = jnp.min(jnp.where(logits == m0, e_iota, E), axis=1, keepdims=True)
    l2 = jnp.where(e_iota == i0, jnp.float32(-1e30), logits)
    m1 = jnp.max(l2, axis=1, keepdims=True)
    i1 = jnp.min(jnp.where(l2 == m1, e_iota, E), axis=1, keepdims=True)

    t = jnp.exp(m1 - m0)
    w0 = 1.0 / (1.0 + t)
    w1g = t / (1.0 + t)
    w_out[...] = jnp.concatenate([w0, w1g], axis=1)
    se_out[...] = jnp.concatenate([i0, i1], axis=1)
    g0_out[...] = w0 + jnp.zeros((N, 16), jnp.float32)
    g1_out[...] = w1g + jnp.zeros((N, 16), jnp.float32)

    oh0 = (e_iota == i0).astype(jnp.float32)
    oh1 = (e_iota == i1).astype(jnp.float32)
    C = 512
    r_i = lax.broadcasted_iota(jnp.int32, (C, C), 0)
    c_i = lax.broadcasted_iota(jnp.int32, (C, C), 1)
    tri = (c_i < r_i).astype(jnp.float32)

    def cumsum_excl(oh):
        run = jnp.zeros((1, E), jnp.float32)
        outs = []
        for c in range(N // C):
            blk = oh[c * C:(c + 1) * C, :]
            outs.append(jnp.dot(tri, blk, preferred_element_type=jnp.float32)
                        + run)
            run = run + jnp.sum(blk, axis=0, keepdims=True)
        return jnp.concatenate(outs, axis=0), run

    r0, tot0 = cumsum_excl(oh0)
    r1, tot1 = cumsum_excl(oh1)
    rank0 = jnp.sum(r0 * oh0, axis=1, keepdims=True)
    rank1 = jnp.sum((r1 + tot0) * oh1, axis=1, keepdims=True)

    counts = tot0 + tot1
    padded = jnp.floor((counts + (T - 1)) * (1.0 / T)) * T
    er = lax.broadcasted_iota(jnp.int32, (E, E), 0)
    ec = lax.broadcasted_iota(jnp.int32, (E, E), 1)
    tri8 = (er < ec).astype(jnp.float32)
    offs = jnp.dot(padded, tri8, preferred_element_type=jnp.float32)

    pos0 = jnp.sum(oh0 * offs, axis=1, keepdims=True) + rank0
    pos1 = jnp.sum(oh1 * offs, axis=1, keepdims=True) + rank1
    p0_out[...] = pos0.astype(jnp.int32)
    p1_out[...] = pos1.astype(jnp.int32)

    b_start = (lax.broadcasted_iota(jnp.int32, (NB, 1), 0) * T).astype(jnp.float32)
    ind = (b_start >= offs) & (b_start < offs + padded)
    e_row = lax.broadcasted_iota(jnp.int32, (NB, E), 1).astype(jnp.float32)
    be = jnp.sum(jnp.where(ind, e_row, 0.0), axis=1, keepdims=True)
    be_out[...] = be.astype(jnp.int32)


def _router_call(x2d, router_w):
    return pl.pallas_call(
        _router_body,
        out_shape=[
            jax.ShapeDtypeStruct((N, TOPK), jnp.float32),
            jax.ShapeDtypeStruct((N, TOPK), jnp.int32),
            jax.ShapeDtypeStruct((N, 1), jnp.int32),
            jax.ShapeDtypeStruct((N, 1), jnp.int32),
            jax.ShapeDtypeStruct((N, 16), jnp.float32),
            jax.ShapeDtypeStruct((N, 16), jnp.float32),
            jax.ShapeDtypeStruct((NB, 1), jnp.int32),
        ],
    )(x2d, router_w)


_GCH = 32
_APT = 2 * N // NW


def _gather_body(x_hbm, pos_hbm, xs_hbm,
                 idx0_v, idx1_v, rows0_v, rows1_v, sem):
    wid = lax.axis_index("s") * NC + lax.axis_index("c")
    abase = wid * _APT
    rbase = (wid % (NW // 2)) * _APT
    idxs = (idx0_v, idx1_v)
    rows = (rows0_v, rows1_v)
    prev = None
    for c in range(_APT // _GCH):
        b = c % 2
        pltpu.sync_copy(pos_hbm.at[pl.ds(abase + c * _GCH, _GCH)], idxs[b])
        pltpu.sync_copy(x_hbm.at[pl.ds(rbase + c * _GCH, _GCH)], rows[b])
        if prev is not None:
            prev.wait()
        prev = pltpu.async_copy(rows[b], xs_hbm.at[idxs[b]], sem)
    prev.wait()


def _gather_call(x2d, pos_all):
    f = pl.kernel(
        _gather_body,
        out_type=jax.ShapeDtypeStruct((A_PAD, D), jnp.float32),
        mesh=plsc.VectorSubcoreMesh(core_axis_name="c", subcore_axis_name="s"),
        scratch_types=[
            pltpu.VMEM((_GCH,), jnp.int32),
            pltpu.VMEM((_GCH,), jnp.int32),
            pltpu.VMEM((_GCH, D), jnp.float32),
            pltpu.VMEM((_GCH, D), jnp.float32),
            pltpu.SemaphoreType.DMA,
        ],
    )
    return f(x2d, pos_all)


def _expert_body(be_ref, xs_ref, w1_ref, w3_ref, w2_ref, ys_ref):
    j = pl.program_id(0)
    g = pl.program_id(1)
    x = xs_ref[...]
    w1e = jnp.squeeze(w1_ref[...], 0)
    w3e = jnp.squeeze(w3_ref[...], 0)
    w2e = jnp.squeeze(w2_ref[...], 0)
    a = lax.dot_general(x, w1e, (((1,), (1,)), ((), ())),
                        preferred_element_type=jnp.float32)
    b = lax.dot_general(x, w3e, (((1,), (1,)), ((), ())),
                        preferred_element_type=jnp.float32)
    h = a * jax.nn.sigmoid(a) * b
    contrib = lax.dot_general(h, w2e, (((1,), (1,)), ((), ())),
                              preferred_element_type=jnp.float32)
    rows = pl.ds(g * T, T)

    @pl.when(j == 0)
    def _():
        ys_ref[rows, :] = contrib

    @pl.when(j > 0)
    def _():
        ys_ref[rows, :] += contrib


def _expert_call(xs, w1, w3, w2, block_expert):
    grid_spec = pltpu.PrefetchScalarGridSpec(
        num_scalar_prefetch=1,
        grid=(NH, NB),
        in_specs=[
            pl.BlockSpec((T, D), lambda j, g, be: (g, 0)),
            pl.BlockSpec((1, HB, D), lambda j, g, be: (be[g], j, 0)),
            pl.BlockSpec((1, HB, D), lambda j, g, be: (be[g], j, 0)),
            pl.BlockSpec((1, D, HB), lambda j, g, be: (be[g], 0, j)),
        ],
        out_specs=pl.BlockSpec((A_PAD, D), lambda j, g, be: (0, 0)),
    )
    return pl.pallas_call(
        _expert_body,
        grid_spec=grid_spec,
        out_shape=jax.ShapeDtypeStruct((A_PAD, D), jnp.float32),
        compiler_params=pltpu.CompilerParams(
            dimension_semantics=("arbitrary", "arbitrary")),
    )(block_expert, xs, w1, w3, w2)


def _combine_body(ys_hbm, p0_hbm, p1_hbm, g0_hbm, g1_hbm, out_hbm,
                  i0_v, i1_v, g0_v, g1_v, a_v, b_v, o_v, sem):
    wid = lax.axis_index("s") * NC + lax.axis_index("c")
    per = N // NW
    base = wid * per
    pltpu.sync_copy(p0_hbm.at[pl.ds(base, per)], i0_v)
    pltpu.sync_copy(p1_hbm.at[pl.ds(base, per)], i1_v)
    pltpu.sync_copy(g0_hbm.at[pl.ds(base, per)], g0_v)
    pltpu.sync_copy(g1_hbm.at[pl.ds(base, per)], g1_v)
    for c in range(per // 16):
        ca = pltpu.async_copy(ys_hbm.at[i0_v.at[pl.ds(c * 16, 16)]], a_v, sem)
        cb = pltpu.async_copy(ys_hbm.at[i1_v.at[pl.ds(c * 16, 16)]], b_v, sem)
        ca.wait()
        cb.wait()

        def row(t, _):
            tt = c * 16 + t
            g0b = g0_v[tt, :]
            g1b = g1_v[tt, :]
            for v in range(D // 16):
                s = pl.ds(v * 16, 16)
                o_v[t, s] = g0b * a_v[t, s] + g1b * b_v[t, s]
            return 0

        lax.fori_loop(0, 16, row, 0)
        pltpu.sync_copy(o_v, out_hbm.at[pl.ds(base + c * 16, 16)])


def _combine_call(ys, pos0, pos1, gate0, gate1):
    f = pl.kernel(
        _combine_body,
        out_type=jax.ShapeDtypeStruct((N, D), jnp.float32),
        mesh=plsc.VectorSubcoreMesh(core_axis_name="c", subcore_axis_name="s"),
        scratch_types=[
            pltpu.VMEM((N // NW,), jnp.int32),
            pltpu.VMEM((N // NW,), jnp.int32),
            pltpu.VMEM((N // NW, 16), jnp.float32),
            pltpu.VMEM((N // NW, 16), jnp.float32),
            pltpu.VMEM((16, D), jnp.float32),
            pltpu.VMEM((16, D), jnp.float32),
            pltpu.VMEM((16, D), jnp.float32),
            pltpu.SemaphoreType.DMA,
        ],
    )
    return f(ys, pos0, pos1, gate0, gate1)


def kernel(x, router_w, w1, w2, w3):
    x2d = x.reshape(N, D)
    (w_top, se, p0, p1, g0, g1, be) = _router_call(x2d, router_w)
    pos_all = jnp.concatenate([p0.reshape(N), p1.reshape(N)])
    xs = _gather_call(x2d, pos_all)
    ys = _expert_call(xs, w1, w3, w2, be.reshape(NB))
    out = _combine_call(ys, p0.reshape(N), p1.reshape(N), g0, g1)
    return (out.reshape(1, N, D),
            w_top.reshape(1, N, TOPK),
            se.reshape(1, N, TOPK))
